# baseline (speedup 1.0000x reference)
import functools

import jax
import jax.numpy as jnp
from jax import lax
from jax.experimental import pallas as pl
from jax.experimental.pallas import tpu as pltpu
from jax.experimental.pallas import tpu_sc as plsc

_N_NODES = 10000
_N_EDGES = 320000
_NODE_DIM = 128
_HID = 32

_NW = 32
_NSEG = 4
_SEG = _N_EDGES // _NSEG
_NRPAD = 81920
_RPW = _NRPAD // _NW
_CR = 80
_NCH = _RPW // _CR
_ACCN = _CR * 128


def _pq_body(nr_ref, w1a_ref, w1b_ref, b1_ref, p_ref, q_ref):
    nr = nr_ref[...]
    p_ref[...] = (
        jnp.dot(nr, w1a_ref[...], preferred_element_type=jnp.float32)
        + b1_ref[...]
    )
    q_ref[...] = jnp.dot(nr, w1b_ref[...], preferred_element_type=jnp.float32)


def _compute_pq(node_rep, w1a, w1b, b1):
    return pl.pallas_call(
        _pq_body,
        out_shape=[
            jax.ShapeDtypeStruct((_N_NODES, _HID), jnp.float32),
            jax.ShapeDtypeStruct((_N_NODES, _HID), jnp.float32),
        ],
    )(node_rep, w1a, w1b, b1)


def _gather_body(p_hbm, q_hbm, src_hbm, dst_hbm, out_a,
                 idx_s, idx_d, rp0, rq0, rp1, rq1, acc0, acc1,
                 sp0, sq0, sp1, sq1, so0, so1):
    wid = lax.axis_index("s") * 2 + lax.axis_index("c")
    for c in range(_NSEG):
        pltpu.sync_copy(src_hbm.at[c, pl.ds(wid * _RPW, _RPW)],
                        idx_s.at[c])
        pltpu.sync_copy(dst_hbm.at[c, pl.ds(wid * _RPW, _RPW)],
                        idx_d.at[c])
    row0 = wid * _RPW
    bufs = ((rp0, rq0, acc0, sp0, sq0, so0),
            (rp1, rq1, acc1, sp1, sq1, so1))

    def fire(j, b):
        rp, rq, _, sp, sq, _ = bufs[b]
        for c in range(_NSEG):
            pltpu.async_copy(
                p_hbm.at[idx_s.at[c, pl.ds(j * _CR, _CR)]], rp.at[c], sp)
            pltpu.async_copy(
                q_hbm.at[idx_d.at[c, pl.ds(j * _CR, _CR)]], rq.at[c], sq)

    def out_slice(j):
        return out_a.at[pl.ds((row0 + j * _CR) * 128, _ACCN)]

    def process(j, b):
        rp, rq, acc, sp, sq, so = bufs[b]
        for c in range(_NSEG):
            pltpu.make_async_copy(
                p_hbm.at[idx_s.at[c, pl.ds(j * _CR, _CR)]],
                rp.at[c], sp).wait()
            pltpu.make_async_copy(
                q_hbm.at[idx_d.at[c, pl.ds(j * _CR, _CR)]],
                rq.at[c], sq).wait()
        for m in range(_CR):
            for c in range(_NSEG):
                for c0 in (0, 16):
                    acc[pl.ds(m * 128 + c * _HID + c0, 16)] = (
                        rp[c, m, pl.ds(c0, 16)] + rq[c, m, pl.ds(c0, 16)]
                    )
        pltpu.async_copy(acc, out_slice(j), so)

    def wait_out(j, b):
        _, _, acc, _, _, so = bufs[b]
        pltpu.make_async_copy(acc, out_slice(j), so).wait()

    fire(0, 0)

    def body(i, carry):
        j0 = 2 * i
        for b in (0, 1):
            j = j0 + b
            fire(j + 1, 1 - b)
            @pl.when(j >= 2)
            def _():
                wait_out(j - 2, b)
            process(j, b)
        return carry

    lax.fori_loop(0, _NCH // 2 - 1, body, 0)
    fire(_NCH - 1, 1)
    wait_out(_NCH - 4, 0)
    process(_NCH - 2, 0)
    wait_out(_NCH - 3, 1)
    process(_NCH - 1, 1)
    wait_out(_NCH - 2, 0)
    wait_out(_NCH - 1, 1)


def _gather_rows(p, q, src_pad, dst_pad):
    mesh = plsc.VectorSubcoreMesh(core_axis_name="c", subcore_axis_name="s")
    k = pl.kernel(
        _gather_body,
        out_type=jax.ShapeDtypeStruct((_NRPAD * 128,), jnp.float32),
        mesh=mesh,
        compiler_params=pltpu.CompilerParams(use_tc_tiling_on_sc=False),
        scratch_types=[
            pltpu.VMEM((_NSEG, _RPW), jnp.int32),
            pltpu.VMEM((_NSEG, _RPW), jnp.int32),
            pltpu.VMEM((_NSEG, _CR, _HID), jnp.float32),
            pltpu.VMEM((_NSEG, _CR, _HID), jnp.float32),
            pltpu.VMEM((_NSEG, _CR, _HID), jnp.float32),
            pltpu.VMEM((_NSEG, _CR, _HID), jnp.float32),
            pltpu.VMEM((_ACCN,), jnp.float32),
            pltpu.VMEM((_ACCN,), jnp.float32),
            pltpu.SemaphoreType.DMA,
            pltpu.SemaphoreType.DMA,
            pltpu.SemaphoreType.DMA,
            pltpu.SemaphoreType.DMA,
            pltpu.SemaphoreType.DMA,
            pltpu.SemaphoreType.DMA,
        ],
    )
    return k(p, q, src_pad, dst_pad)


def _mlp_body(a_ref, w2_ref, b2_ref, w3_ref, b3_ref, out_ref):
    blk = a_ref.shape[0] // 128
    h1 = jnp.maximum(a_ref[...].reshape(blk, 128), 0.0)
    h2 = jnp.dot(h1, w2_ref[...], preferred_element_type=jnp.float32)
    h2 = jnp.maximum(h2 + b2_ref[...], 0.0)
    z = jnp.dot(h2, w3_ref[...], preferred_element_type=jnp.float32)
    zt = jnp.transpose(z) + b3_ref[...]
    out_ref[...] = 1.0 / (1.0 + jnp.exp(-zt))


def _mlp(flat_a, w2p, b2p, w3p, b3):
    blk = 4096
    grid = _NRPAD // blk
    out = pl.pallas_call(
        _mlp_body,
        grid=(grid,),
        in_specs=[
            pl.BlockSpec((blk * 128,), lambda i: (i,)),
            pl.BlockSpec((128, 128), lambda i: (0, 0)),
            pl.BlockSpec((1, 128), lambda i: (0, 0)),
            pl.BlockSpec((128, 4), lambda i: (0, 0)),
            pl.BlockSpec((1, 1), lambda i: (0, 0)),
        ],
        out_specs=pl.BlockSpec((_NSEG, blk), lambda i: (0, i)),
        out_shape=jax.ShapeDtypeStruct((_NSEG, _NRPAD), jnp.float32),
    )(flat_a, w2p, b2p, w3p, b3)
    return out


def kernel(node_rep, edge_index, W1, b1, W2, b2, W3, b3):
    w1a = W1[:_NODE_DIM]
    w1b = W1[_NODE_DIM:]
    p, q = _compute_pq(node_rep, w1a, w1b, b1.reshape(1, _HID))
    pad = _NRPAD - _SEG
    src_pad = jnp.pad(edge_index[0].reshape(_NSEG, _SEG), ((0, 0), (0, pad)))
    dst_pad = jnp.pad(edge_index[1].reshape(_NSEG, _SEG), ((0, 0), (0, pad)))
    rows_a = _gather_rows(p, q, src_pad, dst_pad)
    eye4 = jnp.eye(4, dtype=jnp.float32)
    w2p = jnp.kron(eye4, W2)
    w3p = jnp.kron(eye4, W3)
    b2p = jnp.tile(b2, 4).reshape(1, 128)
    out = _mlp(rows_a, w2p, b2p, w3p, b3.reshape(1, 1))
    return out[:, :_SEG].reshape(_N_EDGES)


# speedup vs baseline: 1.5876x; 1.5876x over previous
import functools

import jax
import jax.numpy as jnp
from jax import lax
from jax.experimental import pallas as pl
from jax.experimental.pallas import tpu as pltpu
from jax.experimental.pallas import tpu_sc as plsc

_N_NODES = 10000
_N_EDGES = 320000
_NODE_DIM = 128
_HID = 32

_NW = 32
_NSEG = 4
_SEG = _N_EDGES // _NSEG
_WPS = _NW // _NSEG
_RPW = _SEG // _WPS
_CR = 80
_NCH = _RPW // _CR


def _pq_body(nr_ref, w1a_ref, w1b_ref, b1_ref, p_ref, q_ref):
    nr = nr_ref[...]
    p_ref[...] = (
        jnp.dot(nr, w1a_ref[...], preferred_element_type=jnp.float32)
        + b1_ref[...]
    )
    q_ref[...] = jnp.dot(nr, w1b_ref[...], preferred_element_type=jnp.float32)


def _compute_pq(node_rep, w1a, w1b, b1):
    return pl.pallas_call(
        _pq_body,
        out_shape=[
            jax.ShapeDtypeStruct((_N_NODES, _HID), jnp.float32),
            jax.ShapeDtypeStruct((_N_NODES, _HID), jnp.float32),
        ],
    )(node_rep, w1a, w1b, b1)


def _gather_body(p_hbm, q_hbm, ei_hbm, out_a,
                 idx_s, idx_d, rp0, rq0, rp1, rq1, acc0, acc1,
                 sp0, sq0, sp1, sq1, so0, so1):
    wid = lax.axis_index("s") * 2 + lax.axis_index("c")
    seg = wid // _WPS
    row0 = (wid % _WPS) * _RPW
    ebase = seg * _SEG + row0
    pltpu.sync_copy(ei_hbm.at[0, pl.ds(ebase, _RPW)], idx_s)
    pltpu.sync_copy(ei_hbm.at[1, pl.ds(ebase, _RPW)], idx_d)
    lane0 = seg * _HID
    bufs = ((rp0, rq0, acc0, sp0, sq0, so0),
            (rp1, rq1, acc1, sp1, sq1, so1))

    def fire(j, b):
        rp, rq, _, sp, sq, _ = bufs[b]
        pltpu.async_copy(p_hbm.at[idx_s.at[pl.ds(j * _CR, _CR)]], rp, sp)
        pltpu.async_copy(q_hbm.at[idx_d.at[pl.ds(j * _CR, _CR)]], rq, sq)

    def out_slice(j):
        return out_a.at[pl.ds(row0 + j * _CR, _CR), pl.ds(lane0, _HID)]

    def process(j, b):
        rp, rq, acc, sp, sq, so = bufs[b]
        pltpu.make_async_copy(
            p_hbm.at[idx_s.at[pl.ds(j * _CR, _CR)]], rp, sp).wait()
        pltpu.make_async_copy(
            q_hbm.at[idx_d.at[pl.ds(j * _CR, _CR)]], rq, sq).wait()
        for r in range(_CR):
            for c0 in (0, 16):
                acc[r, pl.ds(c0, 16)] = (
                    rp[r, pl.ds(c0, 16)] + rq[r, pl.ds(c0, 16)]
                )
        pltpu.async_copy(acc, out_slice(j), so)

    def wait_out(j, b):
        _, _, acc, _, _, so = bufs[b]
        pltpu.make_async_copy(acc, out_slice(j), so).wait()

    fire(0, 0)

    def body(i, carry):
        j0 = 2 * i
        for b in (0, 1):
            j = j0 + b
            fire(j + 1, 1 - b)
            @pl.when(j >= 2)
            def _():
                wait_out(j - 2, b)
            process(j, b)
        return carry

    lax.fori_loop(0, (_NCH - 1) // 2, body, 0)
    wait_out(_NCH - 3, 0)
    process(_NCH - 1, 0)
    wait_out(_NCH - 2, 1)
    wait_out(_NCH - 1, 0)


def _gather_rows(p, q, edge_index):
    mesh = plsc.VectorSubcoreMesh(core_axis_name="c", subcore_axis_name="s")
    k = pl.kernel(
        _gather_body,
        out_type=jax.ShapeDtypeStruct((_SEG, 128), jnp.float32),
        mesh=mesh,
        compiler_params=pltpu.CompilerParams(use_tc_tiling_on_sc=False),
        scratch_types=[
            pltpu.VMEM((_RPW,), jnp.int32),
            pltpu.VMEM((_RPW,), jnp.int32),
            pltpu.VMEM((_CR, _HID), jnp.float32),
            pltpu.VMEM((_CR, _HID), jnp.float32),
            pltpu.VMEM((_CR, _HID), jnp.float32),
            pltpu.VMEM((_CR, _HID), jnp.float32),
            pltpu.VMEM((_CR, _HID), jnp.float32),
            pltpu.VMEM((_CR, _HID), jnp.float32),
            pltpu.SemaphoreType.DMA,
            pltpu.SemaphoreType.DMA,
            pltpu.SemaphoreType.DMA,
            pltpu.SemaphoreType.DMA,
            pltpu.SemaphoreType.DMA,
            pltpu.SemaphoreType.DMA,
        ],
    )
    return k(p, q, edge_index)


def _mlp_body(a_ref, w2_ref, b2_ref, w3_ref, b3_ref, out_ref):
    h1 = jnp.maximum(a_ref[...], 0.0)
    h2 = jnp.dot(h1, w2_ref[...], preferred_element_type=jnp.float32)
    h2 = jnp.maximum(h2 + b2_ref[...], 0.0)
    z = jnp.dot(h2, w3_ref[...], preferred_element_type=jnp.float32)
    zt = jnp.transpose(z) + b3_ref[...]
    out_ref[...] = 1.0 / (1.0 + jnp.exp(-zt))


def _mlp(rows_a, w2p, b2p, w3p, b3):
    blk = 3200
    grid = _SEG // blk
    out = pl.pallas_call(
        _mlp_body,
        grid=(grid,),
        in_specs=[
            pl.BlockSpec((blk, 128), lambda i: (i, 0)),
            pl.BlockSpec((128, 128), lambda i: (0, 0)),
            pl.BlockSpec((1, 128), lambda i: (0, 0)),
            pl.BlockSpec((128, 4), lambda i: (0, 0)),
            pl.BlockSpec((1, 1), lambda i: (0, 0)),
        ],
        out_specs=pl.BlockSpec((_NSEG, blk), lambda i: (0, i)),
        out_shape=jax.ShapeDtypeStruct((_NSEG, _SEG), jnp.float32),
    )(rows_a, w2p, b2p, w3p, b3)
    return out


def kernel(node_rep, edge_index, W1, b1, W2, b2, W3, b3):
    w1a = W1[:_NODE_DIM]
    w1b = W1[_NODE_DIM:]
    p, q = _compute_pq(node_rep, w1a, w1b, b1.reshape(1, _HID))
    rows_a = _gather_rows(p, q, edge_index)
    eye4 = jnp.eye(4, dtype=jnp.float32)
    w2p = jnp.kron(eye4, W2)
    w3p = jnp.kron(eye4, W3)
    b2p = jnp.tile(b2, 4).reshape(1, 128)
    out = _mlp(rows_a, w2p, b2p, w3p, b3.reshape(1, 1))
    return out.reshape(_N_EDGES)
